# baseline (speedup 1.0000x reference)
import functools

import jax
import jax.numpy as jnp
from jax import lax
from jax.experimental import pallas as pl
from jax.experimental.pallas import tpu as pltpu
from jax.experimental.pallas import tpu_sc as plsc

B, L = 4096, 50
N = B * L
SKU_DIM, HID, ITEM_DIM = 64, 128, 128
NPRICE = 100

NW = 32
CHUNK = 128
PER_W = N // NW
NCHUNK = PER_W // CHUNK
NSLOT = 2

_EPS = 1e-5


def _ln(x, g, b):
    mu = jnp.mean(x, axis=-1, keepdims=True)
    var = jnp.mean((x - mu) ** 2, axis=-1, keepdims=True)
    return (x - mu) * lax.rsqrt(var + _EPS) * g + b


_WBLK = 4000
_CBLK = 40


def _prep_body(cat_t, cat_g, cat_b, price_t, price_g, price_b,
               word_t, fc1_w, fc1_b, cp2, w2):
    p2 = jnp.dot(_ln(price_t[...], price_g[...], price_b[...]),
                 fc1_w[256:384, :], preferred_element_type=jnp.float32)
    c2 = jnp.dot(_ln(cat_t[...], cat_g[...], cat_b[...]),
                 fc1_w[128:256, :],
                 preferred_element_type=jnp.float32) + fc1_b[...]
    cp2[...] = (c2[:, None, :] + p2[None, :, :]).reshape(_WBLK, ITEM_DIM)
    w2[...] = jnp.dot(word_t[...], fc1_w[384:512, :],
                      preferred_element_type=jnp.float32)


def _prep(cat_t, cat_g, cat_b, price_t, price_g, price_b, word_t, fc1_w, fc1_b):
    n_cat, n_price, n_word = cat_t.shape[0], price_t.shape[0], word_t.shape[0]
    grid = n_word // _WBLK
    full = lambda shape: pl.BlockSpec(shape, lambda i: (0, 0))
    return pl.pallas_call(
        _prep_body,
        grid=(grid,),
        in_specs=[
            pl.BlockSpec((_CBLK, HID), lambda i: (i, 0)),
            full((1, HID)), full((1, HID)),
            full((n_price, HID)), full((1, HID)), full((1, HID)),
            pl.BlockSpec((_WBLK, HID), lambda i: (i, 0)),
            full((3 * HID + ITEM_DIM, ITEM_DIM)), full((1, ITEM_DIM)),
        ],
        out_specs=[
            pl.BlockSpec((_WBLK, ITEM_DIM), lambda i: (i, 0)),
            pl.BlockSpec((_WBLK, ITEM_DIM), lambda i: (i, 0)),
        ],
        out_shape=[
            jax.ShapeDtypeStruct((n_cat * n_price, ITEM_DIM), jnp.float32),
            jax.ShapeDtypeStruct((n_word, ITEM_DIM), jnp.float32),
        ],
    )(cat_t, cat_g.reshape(1, HID), cat_b.reshape(1, HID),
      price_t, price_g.reshape(1, HID), price_b.reshape(1, HID),
      word_t, fc1_w, fc1_b.reshape(1, ITEM_DIM))


def _gather2_body(cat_idx, price_idx, word_idx, cp2, w2,
                  cp_out, w_out,
                  idx_c, idx_p, idx_w, bufc, bufw, gsems, wsem):
    wid = lax.axis_index("s") * 2 + lax.axis_index("c")
    base = wid * PER_W
    pltpu.sync_copy(cat_idx.at[wid], idx_c)
    pltpu.sync_copy(price_idx.at[wid], idx_p)
    pltpu.sync_copy(word_idx.at[wid], idx_w)

    def xform_cp(k, carry):
        sl = pl.ds(k * 16, 16)
        idx_c[sl] = idx_c[sl] * NPRICE + idx_p[sl]
        return carry

    lax.fori_loop(0, PER_W // 16, xform_cp, 0)

    tabs = [(cp2, idx_c, bufc, cp_out), (w2, idx_w, bufw, w_out)]

    def outer(g, carry):
        c0 = g * NSLOT
        gd = []
        for b in range(NSLOT):
            ds = []
            for table, idx, buf, _ in tabs:
                ds.append(pltpu.async_copy(
                    table.at[idx.at[pl.ds((c0 + b) * CHUNK, CHUNK)]],
                    buf.at[pl.ds(b * CHUNK, CHUNK)], gsems.at[b]))
            gd.append(ds)
        for b in range(NSLOT):
            for d in gd[b]:
                d.wait()
        wd = []
        off = base + c0 * CHUNK
        for _, _, buf, out in tabs:
            wd.append(pltpu.async_copy(
                buf, out.at[pl.ds(off, NSLOT * CHUNK)], wsem))
        for d in wd:
            d.wait()
        return carry

    lax.fori_loop(0, NCHUNK // NSLOT, outer, 0)


def _gather2(cat_idx, price_idx, word_idx, cp2, w2):
    mesh = plsc.VectorSubcoreMesh(core_axis_name="c", subcore_axis_name="s")
    f = functools.partial(
        pl.kernel,
        mesh=mesh,
        out_type=[
            jax.ShapeDtypeStruct((N, ITEM_DIM), jnp.float32),
            jax.ShapeDtypeStruct((N, ITEM_DIM), jnp.float32),
        ],
        scratch_types=[
            pltpu.VMEM((PER_W,), jnp.int32),
            pltpu.VMEM((PER_W,), jnp.int32),
            pltpu.VMEM((PER_W,), jnp.int32),
            pltpu.VMEM((NSLOT * CHUNK, ITEM_DIM), jnp.float32),
            pltpu.VMEM((NSLOT * CHUNK, ITEM_DIM), jnp.float32),
            pltpu.SemaphoreType.DMA((NSLOT,)),
            pltpu.SemaphoreType.DMA,
        ],
    )(_gather2_body)
    return f(cat_idx, price_idx, word_idx, cp2, w2)


def _gather_sku_body(sku_idx, sku_t, sku_out, idx_s, buf64, buf128,
                     gsems, wsems):
    wid = lax.axis_index("s") * 2 + lax.axis_index("c")
    base2 = wid * (PER_W // 2)
    pltpu.sync_copy(sku_idx.at[wid], idx_s)

    def repack(b, r0, carry):
        for r in range(8):
            for c4 in range(4):
                buf128[b, (r0 + r) // 2,
                       pl.ds(64 * ((r0 + r) % 2) + 16 * c4, 16)] = (
                    buf64[b, r0 + r, pl.ds(16 * c4, 16)])
        return carry

    def outer(g, carry):
        gd = []
        for b in range(NSLOT):
            c = g * NSLOT + b
            gd.append(pltpu.async_copy(
                sku_t.at[idx_s.at[pl.ds(c * CHUNK, CHUNK)]],
                buf64.at[b], gsems.at[b]))
        wd = []
        for b in range(NSLOT):
            gd[b].wait()
            lax.fori_loop(0, CHUNK // 8,
                          lambda k, cy: repack(b, k * 8, cy), 0, unroll=2)
            c = g * NSLOT + b
            wd.append(pltpu.async_copy(
                buf128.at[b],
                sku_out.at[pl.ds(base2 + c * (CHUNK // 2), CHUNK // 2)],
                wsems.at[b]))
        for d in wd:
            d.wait()
        return carry

    lax.fori_loop(0, NCHUNK // NSLOT, outer, 0)


def _gather_sku(sku_idx, sku_t):
    mesh = plsc.VectorSubcoreMesh(core_axis_name="c", subcore_axis_name="s")
    f = functools.partial(
        pl.kernel,
        mesh=mesh,
        compiler_params=pltpu.CompilerParams(use_tc_tiling_on_sc=False),
        out_type=jax.ShapeDtypeStruct((N // 2, 2 * SKU_DIM), jnp.float32),
        scratch_types=[
            pltpu.VMEM((PER_W,), jnp.int32),
            pltpu.VMEM((NSLOT, CHUNK, SKU_DIM), jnp.float32),
            pltpu.VMEM((NSLOT, CHUNK // 2, 2 * SKU_DIM), jnp.float32),
            pltpu.SemaphoreType.DMA((NSLOT,)),
            pltpu.SemaphoreType.DMA((NSLOT,)),
        ],
    )(_gather_sku_body)
    return f(sku_idx, sku_t)


_BBLK = 32
_RBLK = _BBLK * L


def _sku_chain(x, sku_g, sku_b, proj_w, proj_b, proj_g, proj_b2, w_s):
    x = _ln(x, sku_g, sku_b)
    x = jnp.dot(x, proj_w, preferred_element_type=jnp.float32) + proj_b
    x = jax.nn.relu(_ln(x, proj_g, proj_b2))
    return jnp.dot(x, w_s, preferred_element_type=jnp.float32)


def _combine_body(skup, cpr, w2r,
                  sku_g, sku_b, proj_w, proj_b, proj_g, proj_b2, w_s, out):
    pair = skup[...]
    args = (sku_g[...], sku_b[...], proj_w[...], proj_b[...],
            proj_g[...], proj_b2[...], w_s[...])
    t_e = _sku_chain(pair[:, :SKU_DIM], *args)
    t_o = _sku_chain(pair[:, SKU_DIM:], *args)
    y = jnp.concatenate([t_e, t_o], axis=1).reshape(_RBLK, ITEM_DIM)
    y = jax.nn.relu(y + cpr[...] + w2r[...])
    out[...] = y.reshape(_BBLK, L, ITEM_DIM)


def _combine(skup, cpr, w2r, sku_g, sku_b,
             proj_w, proj_b, proj_g, proj_b2, w_s):
    grid = N // _RBLK
    row = lambda d: pl.BlockSpec((_RBLK, d), lambda i: (i, 0))
    full = lambda shape: pl.BlockSpec(shape, lambda i: (0, 0))
    return pl.pallas_call(
        _combine_body,
        grid=(grid,),
        in_specs=[
            pl.BlockSpec((_RBLK // 2, ITEM_DIM), lambda i: (i, 0)),
            row(ITEM_DIM), row(ITEM_DIM),
            full((1, SKU_DIM)), full((1, SKU_DIM)),
            full((SKU_DIM, HID)), full((1, HID)), full((1, HID)), full((1, HID)),
            full((HID, ITEM_DIM)),
        ],
        out_specs=pl.BlockSpec((_BBLK, L, ITEM_DIM), lambda i: (i, 0, 0)),
        out_shape=jax.ShapeDtypeStruct((B, L, ITEM_DIM), jnp.float32),
    )(skup, cpr, w2r,
      sku_g.reshape(1, SKU_DIM), sku_b.reshape(1, SKU_DIM),
      proj_w, proj_b.reshape(1, HID), proj_g.reshape(1, HID),
      proj_b2.reshape(1, HID), w_s)


def kernel(sku_id, cat_id, price_id, word_ids, sku_table, sku_ln_g, sku_ln_b,
           proj_W, proj_b, proj_ln_g, proj_ln_b, cat_table, cat_ln_g,
           cat_ln_b, price_table, price_ln_g, price_ln_b, word_table,
           fc1_W, fc1_b):
    cp2, w2 = _prep(cat_table, cat_ln_g, cat_ln_b,
                    price_table, price_ln_g, price_ln_b,
                    word_table, fc1_W, fc1_b)
    shape_ids = lambda a: a.reshape(NW, PER_W).astype(jnp.int32)
    skup = _gather_sku(shape_ids(sku_id), sku_table)
    cpr, w2r = _gather2(shape_ids(cat_id), shape_ids(price_id),
                        shape_ids(word_ids), cp2, w2)
    return _combine(skup, cpr, w2r, sku_ln_g, sku_ln_b,
                    proj_W, proj_b, proj_ln_g, proj_ln_b, fc1_W[:HID, :])


# speedup vs baseline: 1.1170x; 1.1170x over previous
import functools

import jax
import jax.numpy as jnp
from jax import lax
from jax.experimental import pallas as pl
from jax.experimental.pallas import tpu as pltpu
from jax.experimental.pallas import tpu_sc as plsc

B, L = 4096, 50
N = B * L
SKU_DIM, HID, ITEM_DIM = 64, 128, 128
NPRICE = 100

NW = 32
CHUNK = 128
PER_W = N // NW
NCHUNK = PER_W // CHUNK
NSLOT = 2

_EPS = 1e-5


def _ln(x, g, b):
    mu = jnp.mean(x, axis=-1, keepdims=True)
    var = jnp.mean((x - mu) ** 2, axis=-1, keepdims=True)
    return (x - mu) * lax.rsqrt(var + _EPS) * g + b


_WBLK = 4000
_CBLK = 40


def _prep_body(cat_t, cat_g, cat_b, price_t, price_g, price_b,
               word_t, fc1_w, fc1_b, cp2, w2):
    p2 = jnp.dot(_ln(price_t[...], price_g[...], price_b[...]),
                 fc1_w[256:384, :], preferred_element_type=jnp.float32)
    c2 = jnp.dot(_ln(cat_t[...], cat_g[...], cat_b[...]),
                 fc1_w[128:256, :],
                 preferred_element_type=jnp.float32) + fc1_b[...]
    cp2[...] = (c2[:, None, :] + p2[None, :, :]).reshape(_WBLK, ITEM_DIM)
    w2[...] = jnp.dot(word_t[...], fc1_w[384:512, :],
                      preferred_element_type=jnp.float32)


def _prep(cat_t, cat_g, cat_b, price_t, price_g, price_b, word_t, fc1_w, fc1_b):
    n_cat, n_price, n_word = cat_t.shape[0], price_t.shape[0], word_t.shape[0]
    grid = n_word // _WBLK
    full = lambda shape: pl.BlockSpec(shape, lambda i: (0, 0))
    return pl.pallas_call(
        _prep_body,
        grid=(grid,),
        in_specs=[
            pl.BlockSpec((_CBLK, HID), lambda i: (i, 0)),
            full((1, HID)), full((1, HID)),
            full((n_price, HID)), full((1, HID)), full((1, HID)),
            pl.BlockSpec((_WBLK, HID), lambda i: (i, 0)),
            full((3 * HID + ITEM_DIM, ITEM_DIM)), full((1, ITEM_DIM)),
        ],
        out_specs=[
            pl.BlockSpec((_WBLK, ITEM_DIM), lambda i: (i, 0)),
            pl.BlockSpec((_WBLK, ITEM_DIM), lambda i: (i, 0)),
        ],
        out_shape=[
            jax.ShapeDtypeStruct((n_cat * n_price, ITEM_DIM), jnp.float32),
            jax.ShapeDtypeStruct((n_word, ITEM_DIM), jnp.float32),
        ],
    )(cat_t, cat_g.reshape(1, HID), cat_b.reshape(1, HID),
      price_t, price_g.reshape(1, HID), price_b.reshape(1, HID),
      word_t, fc1_w, fc1_b.reshape(1, ITEM_DIM))


def _gather2_body(cat_idx, price_idx, word_idx, cp2, w2,
                  cp_out, w_out,
                  idx_c, idx_p, idx_w, bufc, bufw, gsems, wsem):
    wid = lax.axis_index("s") * 2 + lax.axis_index("c")
    base = wid * PER_W
    pltpu.sync_copy(cat_idx.at[wid], idx_c)
    pltpu.sync_copy(price_idx.at[wid], idx_p)
    pltpu.sync_copy(word_idx.at[wid], idx_w)

    def xform_cp(k, carry):
        sl = pl.ds(k * 16, 16)
        idx_c[sl] = idx_c[sl] * NPRICE + idx_p[sl]
        return carry

    lax.fori_loop(0, PER_W // 16, xform_cp, 0)

    tabs = [(cp2, idx_c, bufc, cp_out), (w2, idx_w, bufw, w_out)]

    def outer(g, carry):
        c0 = g * NSLOT
        gd = []
        for b in range(NSLOT):
            ds = []
            for table, idx, buf, _ in tabs:
                ds.append(pltpu.async_copy(
                    table.at[idx.at[pl.ds((c0 + b) * CHUNK, CHUNK)]],
                    buf.at[pl.ds(b * CHUNK, CHUNK)], gsems.at[b]))
            gd.append(ds)
        for b in range(NSLOT):
            for d in gd[b]:
                d.wait()
        wd = []
        off = base + c0 * CHUNK
        for _, _, buf, out in tabs:
            wd.append(pltpu.async_copy(
                buf, out.at[pl.ds(off, NSLOT * CHUNK)], wsem))
        for d in wd:
            d.wait()
        return carry

    lax.fori_loop(0, NCHUNK // NSLOT, outer, 0)


def _gather2(cat_idx, price_idx, word_idx, cp2, w2):
    mesh = plsc.VectorSubcoreMesh(core_axis_name="c", subcore_axis_name="s")
    f = functools.partial(
        pl.kernel,
        mesh=mesh,
        out_type=[
            jax.ShapeDtypeStruct((N, ITEM_DIM), jnp.float32),
            jax.ShapeDtypeStruct((N, ITEM_DIM), jnp.float32),
        ],
        scratch_types=[
            pltpu.VMEM((PER_W,), jnp.int32),
            pltpu.VMEM((PER_W,), jnp.int32),
            pltpu.VMEM((PER_W,), jnp.int32),
            pltpu.VMEM((NSLOT * CHUNK, ITEM_DIM), jnp.float32),
            pltpu.VMEM((NSLOT * CHUNK, ITEM_DIM), jnp.float32),
            pltpu.SemaphoreType.DMA((NSLOT,)),
            pltpu.SemaphoreType.DMA,
        ],
    )(_gather2_body)
    return f(cat_idx, price_idx, word_idx, cp2, w2)


def _gather_sku_body(sku_idx, sku_t, sku_out, idx_s, buf64, gsems, wsems):
    wid = lax.axis_index("s") * 2 + lax.axis_index("c")
    base = wid * PER_W
    pltpu.sync_copy(sku_idx.at[wid], idx_s)

    def outer(g, carry):
        gd = []
        for b in range(NSLOT):
            c = g * NSLOT + b
            gd.append(pltpu.async_copy(
                sku_t.at[idx_s.at[pl.ds(c * CHUNK, CHUNK)]],
                buf64.at[b], gsems.at[b]))
        wd = []
        for b in range(NSLOT):
            gd[b].wait()
            c = g * NSLOT + b
            wd.append(pltpu.async_copy(
                buf64.at[b], sku_out.at[pl.ds(base + c * CHUNK, CHUNK)],
                wsems.at[b]))
        for d in wd:
            d.wait()
        return carry

    lax.fori_loop(0, NCHUNK // NSLOT, outer, 0)


def _gather_sku(sku_idx, sku_t):
    mesh = plsc.VectorSubcoreMesh(core_axis_name="c", subcore_axis_name="s")
    f = functools.partial(
        pl.kernel,
        mesh=mesh,
        compiler_params=pltpu.CompilerParams(use_tc_tiling_on_sc=False),
        out_type=jax.ShapeDtypeStruct((N, SKU_DIM), jnp.float32),
        scratch_types=[
            pltpu.VMEM((PER_W,), jnp.int32),
            pltpu.VMEM((NSLOT, CHUNK, SKU_DIM), jnp.float32),
            pltpu.SemaphoreType.DMA((NSLOT,)),
            pltpu.SemaphoreType.DMA((NSLOT,)),
        ],
    )(_gather_sku_body)
    return f(sku_idx, sku_t)


_BBLK = 32
_RBLK = _BBLK * L


def _sku_chain(x, sku_g, sku_b, proj_w, proj_b, proj_g, proj_b2, w_s):
    x = _ln(x, sku_g, sku_b)
    x = jnp.dot(x, proj_w, preferred_element_type=jnp.float32) + proj_b
    x = jax.nn.relu(_ln(x, proj_g, proj_b2))
    return jnp.dot(x, w_s, preferred_element_type=jnp.float32)


def _combine_body(skup, cpr, w2r,
                  sku_g, sku_b, proj_w, proj_b, proj_g, proj_b2, w_s, out):
    y = _sku_chain(skup[...], sku_g[...], sku_b[...], proj_w[...],
                   proj_b[...], proj_g[...], proj_b2[...], w_s[...])
    y = jax.nn.relu(y + cpr[...] + w2r[...])
    out[...] = y.reshape(_BBLK, L, ITEM_DIM)


def _combine(skup, cpr, w2r, sku_g, sku_b,
             proj_w, proj_b, proj_g, proj_b2, w_s):
    grid = N // _RBLK
    row = lambda d: pl.BlockSpec((_RBLK, d), lambda i: (i, 0))
    full = lambda shape: pl.BlockSpec(shape, lambda i: (0, 0))
    return pl.pallas_call(
        _combine_body,
        grid=(grid,),
        in_specs=[
            row(SKU_DIM),
            row(ITEM_DIM), row(ITEM_DIM),
            full((1, SKU_DIM)), full((1, SKU_DIM)),
            full((SKU_DIM, HID)), full((1, HID)), full((1, HID)), full((1, HID)),
            full((HID, ITEM_DIM)),
        ],
        out_specs=pl.BlockSpec((_BBLK, L, ITEM_DIM), lambda i: (i, 0, 0)),
        out_shape=jax.ShapeDtypeStruct((B, L, ITEM_DIM), jnp.float32),
    )(skup, cpr, w2r,
      sku_g.reshape(1, SKU_DIM), sku_b.reshape(1, SKU_DIM),
      proj_w, proj_b.reshape(1, HID), proj_g.reshape(1, HID),
      proj_b2.reshape(1, HID), w_s)


def kernel(sku_id, cat_id, price_id, word_ids, sku_table, sku_ln_g, sku_ln_b,
           proj_W, proj_b, proj_ln_g, proj_ln_b, cat_table, cat_ln_g,
           cat_ln_b, price_table, price_ln_g, price_ln_b, word_table,
           fc1_W, fc1_b):
    cp2, w2 = _prep(cat_table, cat_ln_g, cat_ln_b,
                    price_table, price_ln_g, price_ln_b,
                    word_table, fc1_W, fc1_b)
    shape_ids = lambda a: a.reshape(NW, PER_W).astype(jnp.int32)
    cpr, w2r = _gather2(shape_ids(cat_id), shape_ids(price_id),
                        shape_ids(word_ids), cp2, w2)
    skup = _gather_sku(shape_ids(sku_id), sku_table)
    return _combine(skup, cpr, w2r, sku_ln_g, sku_ln_b,
                    proj_W, proj_b, proj_ln_g, proj_ln_b, fc1_W[:HID, :])


# speedup vs baseline: 1.1187x; 1.0015x over previous
import functools

import jax
import jax.numpy as jnp
from jax import lax
from jax.experimental import pallas as pl
from jax.experimental.pallas import tpu as pltpu
from jax.experimental.pallas import tpu_sc as plsc

B, L = 4096, 50
N = B * L
SKU_DIM, HID, ITEM_DIM = 64, 128, 128
NPRICE = 100

NW = 32
CHUNK = 128
PER_W = N // NW
NCHUNK = PER_W // CHUNK
NSLOT = 2

_EPS = 1e-5


def _ln(x, g, b):
    mu = jnp.mean(x, axis=-1, keepdims=True)
    var = jnp.mean((x - mu) ** 2, axis=-1, keepdims=True)
    return (x - mu) * lax.rsqrt(var + _EPS) * g + b


_WBLK = 4000
_CBLK = 40


def _prep_body(cat_t, cat_g, cat_b, price_t, price_g, price_b,
               word_t, fc1_w, fc1_b, cp2, w2):
    p2 = jnp.dot(_ln(price_t[...], price_g[...], price_b[...]),
                 fc1_w[256:384, :], preferred_element_type=jnp.float32)
    c2 = jnp.dot(_ln(cat_t[...], cat_g[...], cat_b[...]),
                 fc1_w[128:256, :],
                 preferred_element_type=jnp.float32) + fc1_b[...]
    cp2[...] = (c2[:, None, :] + p2[None, :, :]).reshape(_WBLK, ITEM_DIM)
    w2[...] = jnp.dot(word_t[...], fc1_w[384:512, :],
                      preferred_element_type=jnp.float32)


def _prep(cat_t, cat_g, cat_b, price_t, price_g, price_b, word_t, fc1_w, fc1_b):
    n_cat, n_price, n_word = cat_t.shape[0], price_t.shape[0], word_t.shape[0]
    grid = n_word // _WBLK
    full = lambda shape: pl.BlockSpec(shape, lambda i: (0, 0))
    return pl.pallas_call(
        _prep_body,
        grid=(grid,),
        in_specs=[
            pl.BlockSpec((_CBLK, HID), lambda i: (i, 0)),
            full((1, HID)), full((1, HID)),
            full((n_price, HID)), full((1, HID)), full((1, HID)),
            pl.BlockSpec((_WBLK, HID), lambda i: (i, 0)),
            full((3 * HID + ITEM_DIM, ITEM_DIM)), full((1, ITEM_DIM)),
        ],
        out_specs=[
            pl.BlockSpec((_WBLK, ITEM_DIM), lambda i: (i, 0)),
            pl.BlockSpec((_WBLK, ITEM_DIM), lambda i: (i, 0)),
        ],
        out_shape=[
            jax.ShapeDtypeStruct((n_cat * n_price, ITEM_DIM), jnp.float32),
            jax.ShapeDtypeStruct((n_word, ITEM_DIM), jnp.float32),
        ],
    )(cat_t, cat_g.reshape(1, HID), cat_b.reshape(1, HID),
      price_t, price_g.reshape(1, HID), price_b.reshape(1, HID),
      word_t, fc1_w, fc1_b.reshape(1, ITEM_DIM))


def _gather2_body(cat_idx, price_idx, word_idx, cp2, w2,
                  cp_out, w_out,
                  idx_c, idx_p, idx_w, bufc, bufw, gsems, wsem):
    wid = lax.axis_index("s") * 2 + lax.axis_index("c")
    base = wid * PER_W
    pltpu.sync_copy(cat_idx.at[wid], idx_c)
    pltpu.sync_copy(price_idx.at[wid], idx_p)
    pltpu.sync_copy(word_idx.at[wid], idx_w)

    def xform_cp(k, carry):
        sl = pl.ds(k * 16, 16)
        idx_c[sl] = idx_c[sl] * NPRICE + idx_p[sl]
        return carry

    lax.fori_loop(0, PER_W // 16, xform_cp, 0)

    tabs = [(cp2, idx_c, bufc, cp_out), (w2, idx_w, bufw, w_out)]

    def outer(g, carry):
        c0 = g * NSLOT
        gd = []
        for b in range(NSLOT):
            ds = []
            for table, idx, buf, _ in tabs:
                ds.append(pltpu.async_copy(
                    table.at[idx.at[pl.ds((c0 + b) * CHUNK, CHUNK)]],
                    buf.at[pl.ds(b * CHUNK, CHUNK)], gsems.at[b]))
            gd.append(ds)
        for b in range(NSLOT):
            for d in gd[b]:
                d.wait()
        wd = []
        off = base + c0 * CHUNK
        for _, _, buf, out in tabs:
            wd.append(pltpu.async_copy(
                buf, out.at[pl.ds(off, NSLOT * CHUNK)], wsem))
        for d in wd:
            d.wait()
        return carry

    lax.fori_loop(0, NCHUNK // NSLOT, outer, 0)


def _gather2(cat_idx, price_idx, word_idx, cp2, w2):
    mesh = plsc.VectorSubcoreMesh(core_axis_name="c", subcore_axis_name="s")
    f = functools.partial(
        pl.kernel,
        mesh=mesh,
        out_type=[
            jax.ShapeDtypeStruct((N, ITEM_DIM), jnp.float32),
            jax.ShapeDtypeStruct((N, ITEM_DIM), jnp.float32),
        ],
        scratch_types=[
            pltpu.VMEM((PER_W,), jnp.int32),
            pltpu.VMEM((PER_W,), jnp.int32),
            pltpu.VMEM((PER_W,), jnp.int32),
            pltpu.VMEM((NSLOT * CHUNK, ITEM_DIM), jnp.float32),
            pltpu.VMEM((NSLOT * CHUNK, ITEM_DIM), jnp.float32),
            pltpu.SemaphoreType.DMA((NSLOT,)),
            pltpu.SemaphoreType.DMA,
        ],
    )(_gather2_body)
    return f(cat_idx, price_idx, word_idx, cp2, w2)


def _gather_sku_body(sku_idx, sku_t, sku_out, idx_s, buf64, gsems, wsems):
    wid = lax.axis_index("s") * 2 + lax.axis_index("c")
    base = wid * PER_W
    pltpu.sync_copy(sku_idx.at[wid], idx_s)

    def outer(g, carry):
        gd = []
        for b in range(NSLOT):
            c = g * NSLOT + b
            gd.append(pltpu.async_copy(
                sku_t.at[idx_s.at[pl.ds(c * CHUNK, CHUNK)]],
                buf64.at[b], gsems.at[b]))
        wd = []
        for b in range(NSLOT):
            gd[b].wait()
            c = g * NSLOT + b
            wd.append(pltpu.async_copy(
                buf64.at[b], sku_out.at[pl.ds(base + c * CHUNK, CHUNK)],
                wsems.at[b]))
        for d in wd:
            d.wait()
        return carry

    lax.fori_loop(0, NCHUNK // NSLOT, outer, 0)


def _gather_sku(sku_idx, sku_t):
    mesh = plsc.VectorSubcoreMesh(core_axis_name="c", subcore_axis_name="s")
    f = functools.partial(
        pl.kernel,
        mesh=mesh,
        compiler_params=pltpu.CompilerParams(use_tc_tiling_on_sc=False, needs_layout_passes=True),
        out_type=jax.ShapeDtypeStruct((N, SKU_DIM), jnp.float32),
        scratch_types=[
            pltpu.VMEM((PER_W,), jnp.int32),
            pltpu.VMEM((NSLOT, CHUNK, SKU_DIM), jnp.float32),
            pltpu.SemaphoreType.DMA((NSLOT,)),
            pltpu.SemaphoreType.DMA((NSLOT,)),
        ],
    )(_gather_sku_body)
    return f(sku_idx, sku_t)


_BBLK = 32
_RBLK = _BBLK * L


def _sku_chain(x, sku_g, sku_b, proj_w, proj_b, proj_g, proj_b2, w_s):
    x = _ln(x, sku_g, sku_b)
    x = jnp.dot(x, proj_w, preferred_element_type=jnp.float32) + proj_b
    x = jax.nn.relu(_ln(x, proj_g, proj_b2))
    return jnp.dot(x, w_s, preferred_element_type=jnp.float32)


def _combine_body(skup, cpr, w2r,
                  sku_g, sku_b, proj_w, proj_b, proj_g, proj_b2, w_s, out):
    y = _sku_chain(skup[...], sku_g[...], sku_b[...], proj_w[...],
                   proj_b[...], proj_g[...], proj_b2[...], w_s[...])
    y = jax.nn.relu(y + cpr[...] + w2r[...])
    out[...] = y.reshape(_BBLK, L, ITEM_DIM)


def _combine(skup, cpr, w2r, sku_g, sku_b,
             proj_w, proj_b, proj_g, proj_b2, w_s):
    grid = N // _RBLK
    row = lambda d: pl.BlockSpec((_RBLK, d), lambda i: (i, 0))
    full = lambda shape: pl.BlockSpec(shape, lambda i: (0, 0))
    return pl.pallas_call(
        _combine_body,
        grid=(grid,),
        in_specs=[
            row(SKU_DIM),
            row(ITEM_DIM), row(ITEM_DIM),
            full((1, SKU_DIM)), full((1, SKU_DIM)),
            full((SKU_DIM, HID)), full((1, HID)), full((1, HID)), full((1, HID)),
            full((HID, ITEM_DIM)),
        ],
        out_specs=pl.BlockSpec((_BBLK, L, ITEM_DIM), lambda i: (i, 0, 0)),
        out_shape=jax.ShapeDtypeStruct((B, L, ITEM_DIM), jnp.float32),
    )(skup, cpr, w2r,
      sku_g.reshape(1, SKU_DIM), sku_b.reshape(1, SKU_DIM),
      proj_w, proj_b.reshape(1, HID), proj_g.reshape(1, HID),
      proj_b2.reshape(1, HID), w_s)


def kernel(sku_id, cat_id, price_id, word_ids, sku_table, sku_ln_g, sku_ln_b,
           proj_W, proj_b, proj_ln_g, proj_ln_b, cat_table, cat_ln_g,
           cat_ln_b, price_table, price_ln_g, price_ln_b, word_table,
           fc1_W, fc1_b):
    cp2, w2 = _prep(cat_table, cat_ln_g, cat_ln_b,
                    price_table, price_ln_g, price_ln_b,
                    word_table, fc1_W, fc1_b)
    shape_ids = lambda a: a.reshape(NW, PER_W).astype(jnp.int32)
    cpr, w2r = _gather2(shape_ids(cat_id), shape_ids(price_id),
                        shape_ids(word_ids), cp2, w2)
    skup = _gather_sku(shape_ids(sku_id), sku_table)
    return _combine(skup, cpr, w2r, sku_ln_g, sku_ln_b,
                    proj_W, proj_b, proj_ln_g, proj_ln_b, fc1_W[:HID, :])
